# baseline (speedup 1.0000x reference)
import functools

import jax
import jax.numpy as jnp
from jax import lax
from jax.experimental import pallas as pl
from jax.experimental.pallas import tpu as pltpu
from jax.experimental.pallas import tpu_sc as plsc

NC = 2
NS = 16
NW = NC * NS
CH = 128
BN_EPS = 1e-5
DEGW = 16
ZROWS = 32
SC0_FRAC = 0.64


def _ceil_to(x, m):
    return (x + m - 1) // m * m


def _zero_vmem_2d(ref, rows, cols):
    z16 = jnp.zeros((16,), jnp.float32)
    per_row = cols // 16

    def st(i, _):
        r = i // per_row
        c = (i % per_row) * 16
        ref[r, pl.ds(c, 16)] = z16
        return 0

    lax.fori_loop(0, rows * per_row, st, 0, unroll=4)


def _zero_acc_slice(zbuf, acc, base, rows, zsem):
    nfull = rows // ZROWS
    rem = rows - nfull * ZROWS
    pieces = [(t * ZROWS, ZROWS) for t in range(nfull)]
    if rem:
        pieces.append((nfull * ZROWS, rem))
    for wave in range(0, len(pieces), 5):
        chunk = pieces[wave:wave + 5]
        descs = []
        for off, sz in chunk:
            descs.append(pltpu.async_copy(
                zbuf.at[pl.ds(0, sz)], acc.at[pl.ds(base + off, sz)], zsem))
        for d in descs:
            d.wait()


def _copyout(acc, out, c, s, n_nodes, rows_per_tile):
    base = s * rows_per_tile
    last = n_nodes - (NS - 1) * rows_per_tile

    @pl.when(s < NS - 1)
    def _():
        pltpu.sync_copy(acc.at[pl.ds(base, rows_per_tile)],
                        out.at[c, pl.ds(base, rows_per_tile)])

    @pl.when(s == NS - 1)
    def _():
        pltpu.sync_copy(acc.at[pl.ds((NS - 1) * rows_per_tile, last)],
                        out.at[c, pl.ds((NS - 1) * rows_per_tile, last)])


def _make_sc_agg(n_nodes, k0, k1, acc_rows, rows_per_tile):
    mesh = plsc.VectorSubcoreMesh(core_axis_name="c", subcore_axis_name="s")

    def body(table, src3, dst3, out, src_v, dst_v, gbuf, zbuf, acc,
             gsem, ssem, zsem):
        c = lax.axis_index("c")
        s = lax.axis_index("s")
        w = c * NS + s
        base = s * rows_per_tile
        kc = jnp.where(c == 0, k0, k1)

        _zero_vmem_2d(zbuf, ZROWS, 128)
        pltpu.sync_copy(src3.at[w], src_v)
        pltpu.sync_copy(dst3.at[w], dst_v)
        _zero_acc_slice(zbuf, acc, base, rows_per_tile, zsem)
        plsc.subcore_barrier()

        def step(j, _):
            pltpu.async_copy(table.at[src_v.at[j]], gbuf, gsem).wait()
            pltpu.async_copy(gbuf, acc.at[dst_v.at[j]], ssem, add=True).wait()
            return 0

        lax.fori_loop(0, kc, step, 0)
        plsc.subcore_barrier()
        _copyout(acc, out, c, s, n_nodes, rows_per_tile)

    kmax = max(k0, k1)
    return pl.kernel(
        body,
        out_type=jax.ShapeDtypeStruct((NC, n_nodes, 128), jnp.float32),
        mesh=mesh,
        scratch_types=[
            pltpu.VMEM((kmax, CH), jnp.int32),
            pltpu.VMEM((kmax, CH), jnp.int32),
            pltpu.VMEM((CH, 128), jnp.float32),
            pltpu.VMEM((ZROWS, 128), jnp.float32),
            pltpu.VMEM_SHARED((acc_rows, 128), jnp.float32),
            pltpu.SemaphoreType.DMA,
            pltpu.SemaphoreType.DMA,
            pltpu.SemaphoreType.DMA,
        ],
    )


def _make_sc_deg(n_nodes, k0, k1, acc_rows, rows_per_tile):
    mesh = plsc.VectorSubcoreMesh(core_axis_name="c", subcore_axis_name="s")

    def body(dst3, out, slab, ones_v, zbuf, acc, ssem, zsem):
        c = lax.axis_index("c")
        s = lax.axis_index("s")
        w = c * NS + s
        base = s * rows_per_tile
        kc = jnp.where(c == 0, k0, k1)

        _zero_vmem_2d(zbuf, ZROWS, DEGW)

        one16 = jnp.ones((16,), jnp.float32)
        def fill(r, _):
            ones_v[r, pl.ds(0, 16)] = one16
            return 0
        lax.fori_loop(0, CH, fill, 0, unroll=4)

        pltpu.sync_copy(dst3.at[w], slab)
        _zero_acc_slice(zbuf, acc, base, rows_per_tile, zsem)
        plsc.subcore_barrier()

        def fire(j, _):
            pltpu.async_copy(ones_v, acc.at[slab.at[j]], ssem, add=True)
            return 0

        def fire_wait(j, _):
            pltpu.async_copy(ones_v, acc.at[slab.at[j]], ssem, add=True)
            pltpu.make_async_copy(ones_v, acc.at[slab.at[j - 8]], ssem).wait()
            return 0

        def drain(j, _):
            pltpu.make_async_copy(ones_v, acc.at[slab.at[j]], ssem).wait()
            return 0

        lax.fori_loop(0, 8, fire, 0)
        lax.fori_loop(8, kc, fire_wait, 0)
        lax.fori_loop(kc - 8, kc, drain, 0)
        plsc.subcore_barrier()
        _copyout(acc, out, c, s, n_nodes, rows_per_tile)

    return pl.kernel(
        body,
        out_type=jax.ShapeDtypeStruct((NC, n_nodes, DEGW), jnp.float32),
        mesh=mesh,
        scratch_types=[
            pltpu.VMEM((max(k0, k1), CH), jnp.int32),
            pltpu.VMEM((CH, DEGW), jnp.float32),
            pltpu.VMEM((ZROWS, DEGW), jnp.float32),
            pltpu.VMEM_SHARED((acc_rows, DEGW), jnp.float32),
            pltpu.SemaphoreType.DMA,
            pltpu.SemaphoreType.DMA,
        ],
    )


BR = 1000


def _tc_prologue_body(degp_ref, x_ref, dinv_ref, hn_ref):
    deg = degp_ref[0, :, 0:1] + degp_ref[1, :, 0:1] + 1.0
    dinv = lax.rsqrt(deg)
    dinv_b = jnp.broadcast_to(dinv, (BR, 128))
    dinv_ref[...] = dinv_b
    hn_ref[...] = x_ref[...] * dinv_b


def _tc_prologue(degp, x, n):
    grid = n // BR
    return pl.pallas_call(
        _tc_prologue_body,
        grid=(grid,),
        in_specs=[
            pl.BlockSpec((NC, BR, DEGW), lambda i: (0, i, 0)),
            pl.BlockSpec((BR, 128), lambda i: (i, 0)),
        ],
        out_specs=[
            pl.BlockSpec((BR, 128), lambda i: (i, 0)),
            pl.BlockSpec((BR, 128), lambda i: (i, 0)),
        ],
        out_shape=[
            jax.ShapeDtypeStruct((n, 128), jnp.float32),
            jax.ShapeDtypeStruct((n, 128), jnp.float32),
        ],
    )(degp, x)


def _bn_relu(y, g_ref, be_ref):
    scale = g_ref[...] * lax.rsqrt(jnp.float32(1.0 + BN_EPS))
    return jnp.maximum(y * scale + be_ref[...], 0.0)


def _tc_layer1_body(p_ref, hn_ref, dinv_ref, w_ref, b_ref, g_ref, be_ref,
                    lo_ref, hi_ref):
    dinv = dinv_ref[...]
    t = dinv * (p_ref[0] + p_ref[1] + hn_ref[...])
    y = jnp.dot(t, w_ref[...], preferred_element_type=jnp.float32) + b_ref[...]
    h = _bn_relu(y, g_ref, be_ref)
    lo_ref[...] = dinv * h[:, :128]
    hi_ref[...] = dinv * h[:, 128:]


def _tc_layer1(p, hn, dinv, w, b, g, be, n):
    grid = n // BR
    full = lambda i: (0, 0)
    return pl.pallas_call(
        _tc_layer1_body,
        grid=(grid,),
        in_specs=[
            pl.BlockSpec((NC, BR, 128), lambda i: (0, i, 0)),
            pl.BlockSpec((BR, 128), lambda i: (i, 0)),
            pl.BlockSpec((BR, 128), lambda i: (i, 0)),
            pl.BlockSpec((128, 256), full),
            pl.BlockSpec((1, 256), full),
            pl.BlockSpec((1, 256), full),
            pl.BlockSpec((1, 256), full),
        ],
        out_specs=[
            pl.BlockSpec((BR, 128), lambda i: (i, 0)),
            pl.BlockSpec((BR, 128), lambda i: (i, 0)),
        ],
        out_shape=[
            jax.ShapeDtypeStruct((n, 128), jnp.float32),
            jax.ShapeDtypeStruct((n, 128), jnp.float32),
        ],
    )(p, hn, dinv, w, b, g, be)


def _tc_mid_body(plo_ref, phi_ref, hnlo_ref, hnhi_ref, dinv_ref,
                 w_ref, b_ref, g_ref, be_ref, lo_ref, hi_ref):
    dinv = dinv_ref[...]
    tlo = dinv * (plo_ref[0] + plo_ref[1] + hnlo_ref[...])
    thi = dinv * (phi_ref[0] + phi_ref[1] + hnhi_ref[...])
    w = w_ref[...]
    y = (jnp.dot(tlo, w[:128, :], preferred_element_type=jnp.float32)
         + jnp.dot(thi, w[128:, :], preferred_element_type=jnp.float32)
         + b_ref[...])
    h = _bn_relu(y, g_ref, be_ref)
    lo_ref[...] = dinv * h[:, :128]
    hi_ref[...] = dinv * h[:, 128:]


def _tc_mid(plo, phi, hnlo, hnhi, dinv, w, b, g, be, n):
    grid = n // BR
    full = lambda i: (0, 0)
    return pl.pallas_call(
        _tc_mid_body,
        grid=(grid,),
        in_specs=[
            pl.BlockSpec((NC, BR, 128), lambda i: (0, i, 0)),
            pl.BlockSpec((NC, BR, 128), lambda i: (0, i, 0)),
            pl.BlockSpec((BR, 128), lambda i: (i, 0)),
            pl.BlockSpec((BR, 128), lambda i: (i, 0)),
            pl.BlockSpec((BR, 128), lambda i: (i, 0)),
            pl.BlockSpec((256, 256), full),
            pl.BlockSpec((1, 256), full),
            pl.BlockSpec((1, 256), full),
            pl.BlockSpec((1, 256), full),
        ],
        out_specs=[
            pl.BlockSpec((BR, 128), lambda i: (i, 0)),
            pl.BlockSpec((BR, 128), lambda i: (i, 0)),
        ],
        out_shape=[
            jax.ShapeDtypeStruct((n, 128), jnp.float32),
            jax.ShapeDtypeStruct((n, 128), jnp.float32),
        ],
    )(plo, phi, hnlo, hnhi, dinv, w, b, g, be)


def _tc_layer4_body(plo_ref, phi_ref, hnlo_ref, hnhi_ref, dinv_ref,
                    w_ref, b_ref, g_ref, be_ref, w5_ref, mn_ref):
    dinv = dinv_ref[...]
    tlo = dinv * (plo_ref[0] + plo_ref[1] + hnlo_ref[...])
    thi = dinv * (phi_ref[0] + phi_ref[1] + hnhi_ref[...])
    w = w_ref[...]
    y = (jnp.dot(tlo, w[:128, :], preferred_element_type=jnp.float32)
         + jnp.dot(thi, w[128:, :], preferred_element_type=jnp.float32)
         + b_ref[...])
    h = _bn_relu(y, g_ref, be_ref)
    m = jnp.dot(h, w5_ref[...], preferred_element_type=jnp.float32)
    mn_ref[...] = dinv * m


def _tc_layer4(plo, phi, hnlo, hnhi, dinv, w, b, g, be, w5, n):
    grid = n // BR
    full = lambda i: (0, 0)
    return pl.pallas_call(
        _tc_layer4_body,
        grid=(grid,),
        in_specs=[
            pl.BlockSpec((NC, BR, 128), lambda i: (0, i, 0)),
            pl.BlockSpec((NC, BR, 128), lambda i: (0, i, 0)),
            pl.BlockSpec((BR, 128), lambda i: (i, 0)),
            pl.BlockSpec((BR, 128), lambda i: (i, 0)),
            pl.BlockSpec((BR, 128), lambda i: (i, 0)),
            pl.BlockSpec((256, 256), full),
            pl.BlockSpec((1, 256), full),
            pl.BlockSpec((1, 256), full),
            pl.BlockSpec((1, 256), full),
            pl.BlockSpec((256, 128), full),
        ],
        out_specs=pl.BlockSpec((BR, 128), lambda i: (i, 0)),
        out_shape=jax.ShapeDtypeStruct((n, 128), jnp.float32),
    )(plo, phi, hnlo, hnhi, dinv, w, b, g, be, w5)


def _tc_final_body(p_ref, mn_ref, dinv_ref, b_ref, out_ref):
    z = dinv_ref[...] * (p_ref[0] + p_ref[1] + mn_ref[...]) + b_ref[...]
    zmax = jnp.max(z, axis=1, keepdims=True)
    ez = jnp.exp(z - zmax)
    lse = jnp.log(jnp.sum(ez, axis=1, keepdims=True))
    out_ref[...] = z - zmax - lse


def _tc_final(p, mn, dinv, b, n):
    grid = n // BR
    full = lambda i: (0, 0)
    return pl.pallas_call(
        _tc_final_body,
        grid=(grid,),
        in_specs=[
            pl.BlockSpec((NC, BR, 128), lambda i: (0, i, 0)),
            pl.BlockSpec((BR, 128), lambda i: (i, 0)),
            pl.BlockSpec((BR, 128), lambda i: (i, 0)),
            pl.BlockSpec((1, 128), full),
        ],
        out_specs=pl.BlockSpec((BR, 128), lambda i: (i, 0)),
        out_shape=jax.ShapeDtypeStruct((n, 128), jnp.float32),
    )(p, mn, dinv, b)


def kernel(x, edge_index, W1, b1, W2, b2, W3, b3, W4, b4, W5, b5,
           g1, be1, g2, be2, g3, be3, g4, be4):
    n = x.shape[0]
    e = edge_index.shape[1]

    rows_per_tile = _ceil_to(-(-(n + 1) // NS), 8)
    acc_rows = rows_per_tile * NS

    e0 = min(e, int(round(e * SC0_FRAC)))
    e1 = e - e0
    k0 = -(-(-(-e0 // NS)) // CH)
    k1 = -(-(-(-e1 // NS)) // CH)
    kmax = max(k0, k1)

    src = edge_index[0].astype(jnp.int32)
    dst = edge_index[1].astype(jnp.int32)

    def _lay(idx, fill, e0_, e1_):
        a0 = jnp.full((NS * k0 * CH,), fill, jnp.int32).at[:e0_].set(
            idx[:e0_]).reshape(NS, k0, CH)
        a1 = jnp.full((NS * k1 * CH,), fill, jnp.int32).at[:e1_].set(
            idx[e0_:]).reshape(NS, k1, CH)
        pad0 = jnp.full((NS, kmax - k0, CH), fill, jnp.int32)
        pad1 = jnp.full((NS, kmax - k1, CH), fill, jnp.int32)
        return jnp.concatenate([
            jnp.concatenate([a0, pad0], axis=1),
            jnp.concatenate([a1, pad1], axis=1)], axis=0)

    src3 = _lay(src, 0, e0, e1)
    dst3 = _lay(dst, n, e0, e1)

    sc_agg = _make_sc_agg(n, k0, k1, acc_rows, rows_per_tile)
    sc_deg = _make_sc_deg(n, k0, k1, acc_rows, rows_per_tile)

    b1r = b1.reshape(1, -1); g1r = g1.reshape(1, -1); be1r = be1.reshape(1, -1)
    b2r = b2.reshape(1, -1); g2r = g2.reshape(1, -1); be2r = be2.reshape(1, -1)
    b3r = b3.reshape(1, -1); g3r = g3.reshape(1, -1); be3r = be3.reshape(1, -1)
    b4r = b4.reshape(1, -1); g4r = g4.reshape(1, -1); be4r = be4.reshape(1, -1)
    b5r = b5.reshape(1, -1)

    degp = sc_deg(dst3)
    dinv, hn1 = _tc_prologue(degp, x, n)

    p1 = sc_agg(hn1, src3, dst3)
    hn2lo, hn2hi = _tc_layer1(p1, hn1, dinv, W1, b1r, g1r, be1r, n)

    p2lo = sc_agg(hn2lo, src3, dst3)
    p2hi = sc_agg(hn2hi, src3, dst3)
    hn3lo, hn3hi = _tc_mid(p2lo, p2hi, hn2lo, hn2hi, dinv, W2, b2r, g2r, be2r, n)

    p3lo = sc_agg(hn3lo, src3, dst3)
    p3hi = sc_agg(hn3hi, src3, dst3)
    hn4lo, hn4hi = _tc_mid(p3lo, p3hi, hn3lo, hn3hi, dinv, W3, b3r, g3r, be3r, n)

    p4lo = sc_agg(hn4lo, src3, dst3)
    p4hi = sc_agg(hn4hi, src3, dst3)
    mn = _tc_layer4(p4lo, p4hi, hn4lo, hn4hi, dinv, W4, b4r, g4r, be4r, W5, n)

    p5 = sc_agg(mn, src3, dst3)
    return _tc_final(p5, mn, dinv, b5r, n)


# speedup vs baseline: 1.0164x; 1.0164x over previous
import jax
import jax.numpy as jnp
from jax import lax
from jax.experimental import pallas as pl
from jax.experimental.pallas import tpu as pltpu
from jax.experimental.pallas import tpu_sc as plsc

NC = 2
NS = 16
NW = NC * NS
CH = 128
BN_EPS = 1e-5
DEGW = 16
ZROWS = 32
SC0_FRAC = 0.62


def _ceil_to(x, m):
    return (x + m - 1) // m * m


def _zero_vmem_2d(ref, rows, cols):
    z16 = jnp.zeros((16,), jnp.float32)
    per_row = cols // 16

    def st(i, _):
        r = i // per_row
        c = (i % per_row) * 16
        ref[r, pl.ds(c, 16)] = z16
        return 0

    lax.fori_loop(0, rows * per_row, st, 0, unroll=4)


def _zero_acc_slice(zbuf, acc, base, rows, zsem):
    nfull = rows // ZROWS
    rem = rows - nfull * ZROWS
    pieces = [(t * ZROWS, ZROWS) for t in range(nfull)]
    if rem:
        pieces.append((nfull * ZROWS, rem))
    for wave in range(0, len(pieces), 5):
        chunk = pieces[wave:wave + 5]
        descs = []
        for off, sz in chunk:
            descs.append(pltpu.async_copy(
                zbuf.at[pl.ds(0, sz)], acc.at[pl.ds(base + off, sz)], zsem))
        for d in descs:
            d.wait()


def _copyout(acc, out, c, s, n_nodes, rows_per_tile):
    base = s * rows_per_tile
    last = n_nodes - (NS - 1) * rows_per_tile

    @pl.when(s < NS - 1)
    def _():
        pltpu.sync_copy(acc.at[pl.ds(base, rows_per_tile)],
                        out.at[c, pl.ds(base, rows_per_tile)])

    @pl.when(s == NS - 1)
    def _():
        pltpu.sync_copy(acc.at[pl.ds((NS - 1) * rows_per_tile, last)],
                        out.at[c, pl.ds((NS - 1) * rows_per_tile, last)])


def _make_sc_agg(n_nodes, k0, k1, acc_rows, rows_per_tile):
    mesh = plsc.VectorSubcoreMesh(core_axis_name="c", subcore_axis_name="s")

    def body(table, src3, dst3, out, src_v, dst_v, gbuf, zbuf, acc,
             gsem, ssem, zsem):
        c = lax.axis_index("c")
        s = lax.axis_index("s")
        w = c * NS + s
        base = s * rows_per_tile
        kc = jnp.where(c == 0, k0, k1)

        _zero_vmem_2d(zbuf, ZROWS, 128)
        pltpu.sync_copy(src3.at[w], src_v)
        pltpu.sync_copy(dst3.at[w], dst_v)
        _zero_acc_slice(zbuf, acc, base, rows_per_tile, zsem)
        plsc.subcore_barrier()

        def step(j, _):
            pltpu.async_copy(table.at[src_v.at[j]], gbuf, gsem).wait()
            pltpu.async_copy(gbuf, acc.at[dst_v.at[j]], ssem, add=True).wait()
            return 0

        lax.fori_loop(0, kc, step, 0)
        plsc.subcore_barrier()
        _copyout(acc, out, c, s, n_nodes, rows_per_tile)

    kmax = max(k0, k1)
    return pl.kernel(
        body,
        out_type=jax.ShapeDtypeStruct((NC, n_nodes, 128), jnp.float32),
        mesh=mesh,
        scratch_types=[
            pltpu.VMEM((kmax, CH), jnp.int32),
            pltpu.VMEM((kmax, CH), jnp.int32),
            pltpu.VMEM((CH, 128), jnp.float32),
            pltpu.VMEM((ZROWS, 128), jnp.float32),
            pltpu.VMEM_SHARED((acc_rows, 128), jnp.float32),
            pltpu.SemaphoreType.DMA,
            pltpu.SemaphoreType.DMA,
            pltpu.SemaphoreType.DMA,
        ],
    )


def _make_sc_deg(n_nodes, k0, k1, acc_rows, rows_per_tile):
    mesh = plsc.VectorSubcoreMesh(core_axis_name="c", subcore_axis_name="s")

    def body(dst3, out, slab, ones_v, zbuf, acc, ssem, zsem):
        c = lax.axis_index("c")
        s = lax.axis_index("s")
        w = c * NS + s
        base = s * rows_per_tile
        kc = jnp.where(c == 0, k0, k1)

        _zero_vmem_2d(zbuf, ZROWS, DEGW)

        one16 = jnp.ones((16,), jnp.float32)
        def fill(r, _):
            ones_v[r, pl.ds(0, 16)] = one16
            return 0
        lax.fori_loop(0, CH, fill, 0, unroll=4)

        pltpu.sync_copy(dst3.at[w], slab)
        _zero_acc_slice(zbuf, acc, base, rows_per_tile, zsem)
        plsc.subcore_barrier()

        def fire(j, _):
            pltpu.async_copy(ones_v, acc.at[slab.at[j]], ssem, add=True)
            return 0

        def fire_wait(j, _):
            pltpu.async_copy(ones_v, acc.at[slab.at[j]], ssem, add=True)
            pltpu.make_async_copy(ones_v, acc.at[slab.at[j - 8]], ssem).wait()
            return 0

        def drain(j, _):
            pltpu.make_async_copy(ones_v, acc.at[slab.at[j]], ssem).wait()
            return 0

        lax.fori_loop(0, 8, fire, 0)
        lax.fori_loop(8, kc, fire_wait, 0)
        lax.fori_loop(kc - 8, kc, drain, 0)
        plsc.subcore_barrier()
        _copyout(acc, out, c, s, n_nodes, rows_per_tile)

    return pl.kernel(
        body,
        out_type=jax.ShapeDtypeStruct((NC, n_nodes, DEGW), jnp.float32),
        mesh=mesh,
        scratch_types=[
            pltpu.VMEM((max(k0, k1), CH), jnp.int32),
            pltpu.VMEM((CH, DEGW), jnp.float32),
            pltpu.VMEM((ZROWS, DEGW), jnp.float32),
            pltpu.VMEM_SHARED((acc_rows, DEGW), jnp.float32),
            pltpu.SemaphoreType.DMA,
            pltpu.SemaphoreType.DMA,
        ],
    )


BR = 1000


def _tc_prologue_body(degp_ref, x_ref, dinv_ref, hn_ref):
    deg = degp_ref[0, :, 0:1] + degp_ref[1, :, 0:1] + 1.0
    dinv = lax.rsqrt(deg)
    dinv_b = jnp.broadcast_to(dinv, (BR, 128))
    dinv_ref[...] = dinv_b
    hn_ref[...] = x_ref[...] * dinv_b


def _tc_prologue(degp, x, n):
    grid = n // BR
    return pl.pallas_call(
        _tc_prologue_body,
        grid=(grid,),
        in_specs=[
            pl.BlockSpec((NC, BR, DEGW), lambda i: (0, i, 0)),
            pl.BlockSpec((BR, 128), lambda i: (i, 0)),
        ],
        out_specs=[
            pl.BlockSpec((BR, 128), lambda i: (i, 0)),
            pl.BlockSpec((BR, 128), lambda i: (i, 0)),
        ],
        out_shape=[
            jax.ShapeDtypeStruct((n, 128), jnp.float32),
            jax.ShapeDtypeStruct((n, 128), jnp.float32),
        ],
    )(degp, x)


def _bn_relu(y, g_ref, be_ref):
    scale = g_ref[...] * lax.rsqrt(jnp.float32(1.0 + BN_EPS))
    return jnp.maximum(y * scale + be_ref[...], 0.0)


def _tc_layer1_body(p_ref, hn_ref, dinv_ref, w_ref, b_ref, g_ref, be_ref,
                    lo_ref, hi_ref):
    dinv = dinv_ref[...]
    t = dinv * (p_ref[0] + p_ref[1] + hn_ref[...])
    y = jnp.dot(t, w_ref[...], preferred_element_type=jnp.float32) + b_ref[...]
    h = _bn_relu(y, g_ref, be_ref)
    lo_ref[...] = dinv * h[:, :128]
    hi_ref[...] = dinv * h[:, 128:]


def _tc_layer1(p, hn, dinv, w, b, g, be, n):
    grid = n // BR
    full = lambda i: (0, 0)
    return pl.pallas_call(
        _tc_layer1_body,
        grid=(grid,),
        in_specs=[
            pl.BlockSpec((NC, BR, 128), lambda i: (0, i, 0)),
            pl.BlockSpec((BR, 128), lambda i: (i, 0)),
            pl.BlockSpec((BR, 128), lambda i: (i, 0)),
            pl.BlockSpec((128, 256), full),
            pl.BlockSpec((1, 256), full),
            pl.BlockSpec((1, 256), full),
            pl.BlockSpec((1, 256), full),
        ],
        out_specs=[
            pl.BlockSpec((BR, 128), lambda i: (i, 0)),
            pl.BlockSpec((BR, 128), lambda i: (i, 0)),
        ],
        out_shape=[
            jax.ShapeDtypeStruct((n, 128), jnp.float32),
            jax.ShapeDtypeStruct((n, 128), jnp.float32),
        ],
    )(p, hn, dinv, w, b, g, be)


def _tc_mid_body(plo_ref, phi_ref, hnlo_ref, hnhi_ref, dinv_ref,
                 w_ref, b_ref, g_ref, be_ref, lo_ref, hi_ref):
    dinv = dinv_ref[...]
    tlo = dinv * (plo_ref[0] + plo_ref[1] + hnlo_ref[...])
    thi = dinv * (phi_ref[0] + phi_ref[1] + hnhi_ref[...])
    w = w_ref[...]
    y = (jnp.dot(tlo, w[:128, :], preferred_element_type=jnp.float32)
         + jnp.dot(thi, w[128:, :], preferred_element_type=jnp.float32)
         + b_ref[...])
    h = _bn_relu(y, g_ref, be_ref)
    lo_ref[...] = dinv * h[:, :128]
    hi_ref[...] = dinv * h[:, 128:]


def _tc_mid(plo, phi, hnlo, hnhi, dinv, w, b, g, be, n):
    grid = n // BR
    full = lambda i: (0, 0)
    return pl.pallas_call(
        _tc_mid_body,
        grid=(grid,),
        in_specs=[
            pl.BlockSpec((NC, BR, 128), lambda i: (0, i, 0)),
            pl.BlockSpec((NC, BR, 128), lambda i: (0, i, 0)),
            pl.BlockSpec((BR, 128), lambda i: (i, 0)),
            pl.BlockSpec((BR, 128), lambda i: (i, 0)),
            pl.BlockSpec((BR, 128), lambda i: (i, 0)),
            pl.BlockSpec((256, 256), full),
            pl.BlockSpec((1, 256), full),
            pl.BlockSpec((1, 256), full),
            pl.BlockSpec((1, 256), full),
        ],
        out_specs=[
            pl.BlockSpec((BR, 128), lambda i: (i, 0)),
            pl.BlockSpec((BR, 128), lambda i: (i, 0)),
        ],
        out_shape=[
            jax.ShapeDtypeStruct((n, 128), jnp.float32),
            jax.ShapeDtypeStruct((n, 128), jnp.float32),
        ],
    )(plo, phi, hnlo, hnhi, dinv, w, b, g, be)


def _tc_layer4_body(plo_ref, phi_ref, hnlo_ref, hnhi_ref, dinv_ref,
                    w_ref, b_ref, g_ref, be_ref, w5_ref, mn_ref):
    dinv = dinv_ref[...]
    tlo = dinv * (plo_ref[0] + plo_ref[1] + hnlo_ref[...])
    thi = dinv * (phi_ref[0] + phi_ref[1] + hnhi_ref[...])
    w = w_ref[...]
    y = (jnp.dot(tlo, w[:128, :], preferred_element_type=jnp.float32)
         + jnp.dot(thi, w[128:, :], preferred_element_type=jnp.float32)
         + b_ref[...])
    h = _bn_relu(y, g_ref, be_ref)
    m = jnp.dot(h, w5_ref[...], preferred_element_type=jnp.float32)
    mn_ref[...] = dinv * m


def _tc_layer4(plo, phi, hnlo, hnhi, dinv, w, b, g, be, w5, n):
    grid = n // BR
    full = lambda i: (0, 0)
    return pl.pallas_call(
        _tc_layer4_body,
        grid=(grid,),
        in_specs=[
            pl.BlockSpec((NC, BR, 128), lambda i: (0, i, 0)),
            pl.BlockSpec((NC, BR, 128), lambda i: (0, i, 0)),
            pl.BlockSpec((BR, 128), lambda i: (i, 0)),
            pl.BlockSpec((BR, 128), lambda i: (i, 0)),
            pl.BlockSpec((BR, 128), lambda i: (i, 0)),
            pl.BlockSpec((256, 256), full),
            pl.BlockSpec((1, 256), full),
            pl.BlockSpec((1, 256), full),
            pl.BlockSpec((1, 256), full),
            pl.BlockSpec((256, 128), full),
        ],
        out_specs=pl.BlockSpec((BR, 128), lambda i: (i, 0)),
        out_shape=jax.ShapeDtypeStruct((n, 128), jnp.float32),
    )(plo, phi, hnlo, hnhi, dinv, w, b, g, be, w5)


def _tc_final_body(p_ref, mn_ref, dinv_ref, b_ref, out_ref):
    z = dinv_ref[...] * (p_ref[0] + p_ref[1] + mn_ref[...]) + b_ref[...]
    zmax = jnp.max(z, axis=1, keepdims=True)
    ez = jnp.exp(z - zmax)
    lse = jnp.log(jnp.sum(ez, axis=1, keepdims=True))
    out_ref[...] = z - zmax - lse


def _tc_final(p, mn, dinv, b, n):
    grid = n // BR
    full = lambda i: (0, 0)
    return pl.pallas_call(
        _tc_final_body,
        grid=(grid,),
        in_specs=[
            pl.BlockSpec((NC, BR, 128), lambda i: (0, i, 0)),
            pl.BlockSpec((BR, 128), lambda i: (i, 0)),
            pl.BlockSpec((BR, 128), lambda i: (i, 0)),
            pl.BlockSpec((1, 128), full),
        ],
        out_specs=pl.BlockSpec((BR, 128), lambda i: (i, 0)),
        out_shape=jax.ShapeDtypeStruct((n, 128), jnp.float32),
    )(p, mn, dinv, b)


def kernel(x, edge_index, W1, b1, W2, b2, W3, b3, W4, b4, W5, b5,
           g1, be1, g2, be2, g3, be3, g4, be4):
    n = x.shape[0]
    e = edge_index.shape[1]

    rows_per_tile = _ceil_to(-(-(n + 1) // NS), 8)
    acc_rows = rows_per_tile * NS

    e0 = min(e, int(round(e * SC0_FRAC)))
    e1 = e - e0
    k0 = -(-(-(-e0 // NS)) // CH)
    k1 = -(-(-(-e1 // NS)) // CH)
    kmax = max(k0, k1)

    src = edge_index[0].astype(jnp.int32)
    dst = edge_index[1].astype(jnp.int32)

    def _lay(idx, fill, e0_, e1_):
        a0 = jnp.full((NS * k0 * CH,), fill, jnp.int32).at[:e0_].set(
            idx[:e0_]).reshape(NS, k0, CH)
        a1 = jnp.full((NS * k1 * CH,), fill, jnp.int32).at[:e1_].set(
            idx[e0_:]).reshape(NS, k1, CH)
        pad0 = jnp.full((NS, kmax - k0, CH), fill, jnp.int32)
        pad1 = jnp.full((NS, kmax - k1, CH), fill, jnp.int32)
        return jnp.concatenate([
            jnp.concatenate([a0, pad0], axis=1),
            jnp.concatenate([a1, pad1], axis=1)], axis=0)

    src3 = _lay(src, 0, e0, e1)
    dst3 = _lay(dst, n, e0, e1)

    sc_agg = _make_sc_agg(n, k0, k1, acc_rows, rows_per_tile)
    sc_deg = _make_sc_deg(n, k0, k1, acc_rows, rows_per_tile)

    b1r = b1.reshape(1, -1); g1r = g1.reshape(1, -1); be1r = be1.reshape(1, -1)
    b2r = b2.reshape(1, -1); g2r = g2.reshape(1, -1); be2r = be2.reshape(1, -1)
    b3r = b3.reshape(1, -1); g3r = g3.reshape(1, -1); be3r = be3.reshape(1, -1)
    b4r = b4.reshape(1, -1); g4r = g4.reshape(1, -1); be4r = be4.reshape(1, -1)
    b5r = b5.reshape(1, -1)

    degp = sc_deg(dst3)
    dinv, hn1 = _tc_prologue(degp, x, n)

    p1 = sc_agg(hn1, src3, dst3)
    hn2lo, hn2hi = _tc_layer1(p1, hn1, dinv, W1, b1r, g1r, be1r, n)

    p2lo = sc_agg(hn2lo, src3, dst3)
    p2hi = sc_agg(hn2hi, src3, dst3)
    hn3lo, hn3hi = _tc_mid(p2lo, p2hi, hn2lo, hn2hi, dinv, W2, b2r, g2r, be2r, n)

    p3lo = sc_agg(hn3lo, src3, dst3)
    p3hi = sc_agg(hn3hi, src3, dst3)
    hn4lo, hn4hi = _tc_mid(p3lo, p3hi, hn3lo, hn3hi, dinv, W3, b3r, g3r, be3r, n)

    p4lo = sc_agg(hn4lo, src3, dst3)
    p4hi = sc_agg(hn4hi, src3, dst3)
    mn = _tc_layer4(p4lo, p4hi, hn4lo, hn4hi, dinv, W4, b4r, g4r, be4r, W5, n)

    p5 = sc_agg(mn, src3, dst3)
    return _tc_final(p5, mn, dinv, b5r, n)
